# baseline (speedup 1.0000x reference)
import numpy as np
import jax
import jax.numpy as jnp
from jax import lax
from jax.experimental import pallas as pl
from jax.experimental.pallas import tpu as pltpu
from jax.experimental.pallas import tpu_sc as plsc

_TOP_K = 512
_NC, _NS, _L = 2, 16, 16
_NW = _NC * _NS
_NB = 256
_HIST = _L * _NB


def _sc_body(f_hbm, d_hbm, out_hbm, fbuf, dbuf, ubuf, cbuf, hist,
             sem_in, sem_out):
    B, D = f_hbm.shape
    rows_per_w = B // _NW
    wid = lax.axis_index("s") * _NC + lax.axis_index("c")
    base = wid * rows_per_w
    laneseq = lax.iota(jnp.int32, _L)
    laneoff = laneseq * _NB
    ones = jnp.ones((_L,), jnp.int32)
    zeros_v = jnp.zeros((_L,), jnp.int32)
    v15 = jnp.full((_L,), _L - 1, jnp.int32)

    @plsc.parallel_loop(0, _HIST, step=_L)
    def _(i):
        hist[pl.ds(i, _L)] = zeros_v

    def scan_pass(C_v):
        init = (zeros_v, jnp.full((_L,), -1, jnp.int32), zeros_v, zeros_v)

        @plsc.parallel_loop(0, _NB, step=_L, carry=init)
        def scan(j, carry):
            run, bsel, nin, nbelow = carry
            acc = zeros_v
            for l in range(_L):
                s = pl.ds(l * _NB + j, _L)
                acc = acc + hist[s]
                hist[s] = zeros_v
            cum = jnp.cumsum(acc)
            inc = run + cum
            m = inc > C_v
            cnt = plsc.all_reduce_population_count(m)
            lane = _L - cnt
            lane_c = jnp.minimum(lane, v15)
            cnt_at = jnp.take_along_axis(acc, lane_c, axis=0)
            cum_at = jnp.take_along_axis(cum, lane_c, axis=0)
            first = jnp.logical_and(cnt > 0, bsel < 0)
            bsel = jnp.where(first, lane + j, bsel)
            nin = jnp.where(first, cnt_at, nin)
            nbelow = jnp.where(first, run + cum_at - cnt_at, nbelow)
            run = run + jnp.take_along_axis(cum, v15, axis=0)
            return run, bsel, nin, nbelow

        _, bsel, nin, nbelow = scan
        return bsel, nin, nbelow

    pltpu.async_copy(f_hbm.at[base], fbuf.at[pl.ds(0, D)], sem_in.at[0])
    pltpu.async_copy(d_hbm.at[base], dbuf.at[pl.ds(0, D)], sem_in.at[0])

    def process_row(row, r, sb, nb, si, so, ni):
        @pl.when(r + 1 < rows_per_w)
        def _():
            pltpu.async_copy(f_hbm.at[row + 1], fbuf.at[pl.ds(nb, D)],
                             sem_in.at[ni])
            pltpu.async_copy(d_hbm.at[row + 1], dbuf.at[pl.ds(nb, D)],
                             sem_in.at[ni])

        pltpu.make_async_copy(f_hbm.at[row], fbuf.at[pl.ds(sb, D)],
                              sem_in.at[si]).wait()
        pltpu.make_async_copy(d_hbm.at[row], dbuf.at[pl.ds(sb, D)],
                              sem_in.at[si]).wait()

        @plsc.parallel_loop(0, D, step=_L, unroll=4)
        def _(i):
            s = pl.ds(sb + i, _L)
            f = fbuf[s]
            dd = dbuf[s]
            m = (f * dd) * dd
            bits = lax.bitcast_convert_type(m, jnp.int32)
            ui = bits ^ ((bits >> 31) | jnp.int32(-2147483648))
            u = lax.bitcast_convert_type(ui, jnp.float32)
            ubuf[s] = u
            dig = (ui >> 24) & jnp.int32(0xFF)
            plsc.addupdate_scatter(hist, [laneoff + dig], ones)

        n_cur = jnp.full((_L,), D, jnp.int32)
        k_cur = jnp.full((_L,), _TOP_K, jnp.int32)

        bsel, nin, nbelow = scan_pass(n_cur - k_cur)
        k_cur = k_cur - (n_cur - nbelow - nin)
        n_cur = nin
        prefix = lax.convert_element_type(bsel, jnp.uint32)

        @plsc.parallel_loop(0, D, step=_L, unroll=4, carry=jnp.int32(0))
        def scat2(i, off, prefix=prefix):
            u = lax.bitcast_convert_type(ubuf[pl.ds(sb + i, _L)],
                                         jnp.uint32)
            msk = (u >> np.uint32(24)) == prefix
            dig = lax.convert_element_type(
                (u >> np.uint32(16)) & np.uint32(0xFF), jnp.int32)
            plsc.addupdate_scatter(hist, [laneoff + dig], ones, mask=msk)
            plsc.store_compressed(cbuf.at[pl.ds(off, _L)], u, mask=msk)
            return off + jnp.sum(msk.astype(jnp.int32))

        n1_s = jnp.max(nin)
        n1_v = nin

        bsel, nin, nbelow = scan_pass(n_cur - k_cur)
        k_cur = k_cur - (n_cur - nbelow - nin)
        n_cur = nin
        prefix = (prefix << np.uint32(8)) | lax.convert_element_type(
            bsel, jnp.uint32)

        @plsc.parallel_loop(0, ((n1_s + _L - 1) // _L) * _L, step=_L)
        def _(j, prefix=prefix, n1_v=n1_v):
            u = cbuf[pl.ds(j, _L)]
            valid = (laneseq + j) < n1_v
            msk = jnp.logical_and(valid, (u >> np.uint32(16)) == prefix)
            dig = lax.convert_element_type(
                (u >> np.uint32(8)) & np.uint32(0xFF), jnp.int32)
            plsc.addupdate_scatter(hist, [laneoff + dig], ones, mask=msk)

        bsel, nin, nbelow = scan_pass(n_cur - k_cur)
        k_cur = k_cur - (n_cur - nbelow - nin)
        n_cur = nin
        prefix = (prefix << np.uint32(8)) | lax.convert_element_type(
            bsel, jnp.uint32)

        @plsc.parallel_loop(0, ((n1_s + _L - 1) // _L) * _L, step=_L)
        def _(j, prefix=prefix, n1_v=n1_v):
            u = cbuf[pl.ds(j, _L)]
            valid = (laneseq + j) < n1_v
            msk = jnp.logical_and(valid, (u >> np.uint32(8)) == prefix)
            dig = lax.convert_element_type(u & np.uint32(0xFF), jnp.int32)
            plsc.addupdate_scatter(hist, [laneoff + dig], ones, mask=msk)

        bsel, _, _ = scan_pass(n_cur - k_cur)
        thresh = (prefix << np.uint32(8)) | lax.convert_element_type(
            bsel, jnp.uint32)

        @plsc.parallel_loop(0, D, step=_L, unroll=4)
        def _(i, thresh=thresh):
            s = pl.ds(sb + i, _L)
            u = lax.bitcast_convert_type(ubuf[s], jnp.uint32)
            p = fbuf[s] * dbuf[s]
            ubuf[s] = jnp.where(u >= thresh, p, jnp.float32(0.0))

        pltpu.sync_copy(ubuf.at[pl.ds(sb, D)], out_hbm.at[row])

    def pair_step(q, _):
        r0 = 2 * q
        process_row(base + r0, r0, 0, D, 0, 0, 1)
        process_row(base + r0 + 1, r0 + 1, D, 0, 1, 1, 0)
        return 0

    lax.fori_loop(0, rows_per_w // 2, pair_step, 0)


def kernel(f_x, dead_latents):
    B, D = f_x.shape
    mesh = plsc.VectorSubcoreMesh(core_axis_name="c", subcore_axis_name="s",
                                  num_cores=_NC, num_subcores=_NS)
    run = pl.kernel(
        _sc_body,
        out_type=jax.ShapeDtypeStruct((B, D), jnp.float32),
        mesh=mesh,
        compiler_params=pltpu.CompilerParams(needs_layout_passes=False),
        scratch_types=[
            pltpu.VMEM((2 * D,), jnp.float32),
            pltpu.VMEM((2 * D,), jnp.float32),
            pltpu.VMEM((2 * D,), jnp.float32),
            pltpu.VMEM((D + _L,), jnp.uint32),
            pltpu.VMEM((_HIST,), jnp.int32),
            pltpu.SemaphoreType.DMA((2,)),
            pltpu.SemaphoreType.DMA((2,)),
        ],
    )
    return run(f_x, dead_latents)


# speedup vs baseline: 1.0766x; 1.0766x over previous
import numpy as np
import jax
import jax.numpy as jnp
from jax import lax
from jax.experimental import pallas as pl
from jax.experimental.pallas import tpu as pltpu
from jax.experimental.pallas import tpu_sc as plsc

_TOP_K = 512
_NC, _NS, _L = 2, 16, 16
_NW = _NC * _NS
_NB = 256
_HIST = _L * _NB


def _sc_body(f_hbm, d_hbm, out_hbm, fbuf, dbuf, ubuf, cbuf, hist,
             sem_in, sem_out):
    B, D = f_hbm.shape
    rows_per_w = B // _NW
    wid = lax.axis_index("s") * _NC + lax.axis_index("c")
    base = wid * rows_per_w
    laneseq = lax.iota(jnp.int32, _L)
    laneoff = laneseq * _NB
    ones = jnp.ones((_L,), jnp.int32)
    zeros_v = jnp.zeros((_L,), jnp.int32)
    v15 = jnp.full((_L,), _L - 1, jnp.int32)

    @plsc.parallel_loop(0, _HIST, step=_L)
    def _(i):
        hist[pl.ds(i, _L)] = zeros_v

    def scan_pass(C_v):
        init = (zeros_v, jnp.full((_L,), -1, jnp.int32), zeros_v, zeros_v)

        @plsc.parallel_loop(0, _NB, step=_L, carry=init)
        def scan(j, carry):
            run, bsel, nin, nbelow = carry
            acc = zeros_v
            for l in range(_L):
                s = pl.ds(l * _NB + j, _L)
                acc = acc + hist[s]
                hist[s] = zeros_v
            cum = jnp.cumsum(acc)
            inc = run + cum
            m = inc > C_v
            cnt = plsc.all_reduce_population_count(m)
            lane = _L - cnt
            lane_c = jnp.minimum(lane, v15)
            cnt_at = jnp.take_along_axis(acc, lane_c, axis=0)
            cum_at = jnp.take_along_axis(cum, lane_c, axis=0)
            first = jnp.logical_and(cnt > 0, bsel < 0)
            bsel = jnp.where(first, lane + j, bsel)
            nin = jnp.where(first, cnt_at, nin)
            nbelow = jnp.where(first, run + cum_at - cnt_at, nbelow)
            run = run + jnp.take_along_axis(cum, v15, axis=0)
            return run, bsel, nin, nbelow

        _, bsel, nin, nbelow = scan
        return bsel, nin, nbelow

    pltpu.async_copy(f_hbm.at[base], fbuf.at[pl.ds(0, D)], sem_in.at[0])
    pltpu.async_copy(d_hbm.at[base], dbuf.at[pl.ds(0, D)], sem_in.at[0])

    def process_row(row, r, sb, nb, si, so, ni):
        @pl.when(r + 1 < rows_per_w)
        def _():
            pltpu.async_copy(f_hbm.at[row + 1], fbuf.at[pl.ds(nb, D)],
                             sem_in.at[ni])
            pltpu.async_copy(d_hbm.at[row + 1], dbuf.at[pl.ds(nb, D)],
                             sem_in.at[ni])

        pltpu.make_async_copy(f_hbm.at[row], fbuf.at[pl.ds(sb, D)],
                              sem_in.at[si]).wait()
        pltpu.make_async_copy(d_hbm.at[row], dbuf.at[pl.ds(sb, D)],
                              sem_in.at[si]).wait()

        @pl.when(r >= 2)
        def _():
            pltpu.make_async_copy(ubuf.at[pl.ds(sb, D)], out_hbm.at[row - 2],
                                  sem_out.at[so]).wait()

        @plsc.parallel_loop(0, D, step=_L, unroll=4)
        def _(i):
            s = pl.ds(sb + i, _L)
            f = fbuf[s]
            dd = dbuf[s]
            m = (f * dd) * dd
            bits = lax.bitcast_convert_type(m, jnp.int32)
            ui = bits ^ ((bits >> 31) | jnp.int32(-2147483648))
            u = lax.bitcast_convert_type(ui, jnp.float32)
            ubuf[s] = u
            dig = (ui >> 24) & jnp.int32(0xFF)
            plsc.addupdate_scatter(hist, [laneoff + dig], ones)

        n_cur = jnp.full((_L,), D, jnp.int32)
        k_cur = jnp.full((_L,), _TOP_K, jnp.int32)

        bsel, nin, nbelow = scan_pass(n_cur - k_cur)
        k_cur = k_cur - (n_cur - nbelow - nin)
        n_cur = nin
        prefix = lax.convert_element_type(bsel, jnp.uint32)

        @plsc.parallel_loop(0, D, step=_L, unroll=4, carry=jnp.int32(0))
        def scat2(i, off, prefix=prefix):
            u = lax.bitcast_convert_type(ubuf[pl.ds(sb + i, _L)],
                                         jnp.uint32)
            msk = (u >> np.uint32(24)) == prefix
            dig = lax.convert_element_type(
                (u >> np.uint32(16)) & np.uint32(0xFF), jnp.int32)
            plsc.addupdate_scatter(hist, [laneoff + dig], ones, mask=msk)
            plsc.store_compressed(cbuf.at[pl.ds(off, _L)], u, mask=msk)
            return off + jnp.sum(msk.astype(jnp.int32))

        n1_s = jnp.max(nin)
        n1_v = nin

        bsel, nin, nbelow = scan_pass(n_cur - k_cur)
        k_cur = k_cur - (n_cur - nbelow - nin)
        n_cur = nin
        prefix = (prefix << np.uint32(8)) | lax.convert_element_type(
            bsel, jnp.uint32)

        @plsc.parallel_loop(0, ((n1_s + _L - 1) // _L) * _L, step=_L)
        def _(j, prefix=prefix, n1_v=n1_v):
            u = cbuf[pl.ds(j, _L)]
            valid = (laneseq + j) < n1_v
            msk = jnp.logical_and(valid, (u >> np.uint32(16)) == prefix)
            dig = lax.convert_element_type(
                (u >> np.uint32(8)) & np.uint32(0xFF), jnp.int32)
            plsc.addupdate_scatter(hist, [laneoff + dig], ones, mask=msk)

        bsel, nin, nbelow = scan_pass(n_cur - k_cur)
        k_cur = k_cur - (n_cur - nbelow - nin)
        n_cur = nin
        prefix = (prefix << np.uint32(8)) | lax.convert_element_type(
            bsel, jnp.uint32)

        @plsc.parallel_loop(0, ((n1_s + _L - 1) // _L) * _L, step=_L)
        def _(j, prefix=prefix, n1_v=n1_v):
            u = cbuf[pl.ds(j, _L)]
            valid = (laneseq + j) < n1_v
            msk = jnp.logical_and(valid, (u >> np.uint32(8)) == prefix)
            dig = lax.convert_element_type(u & np.uint32(0xFF), jnp.int32)
            plsc.addupdate_scatter(hist, [laneoff + dig], ones, mask=msk)

        bsel, _, _ = scan_pass(n_cur - k_cur)
        thresh = (prefix << np.uint32(8)) | lax.convert_element_type(
            bsel, jnp.uint32)

        @plsc.parallel_loop(0, D, step=_L, unroll=4)
        def _(i, thresh=thresh):
            s = pl.ds(sb + i, _L)
            u = lax.bitcast_convert_type(ubuf[s], jnp.uint32)
            p = fbuf[s] * dbuf[s]
            ubuf[s] = jnp.where(u >= thresh, p, jnp.float32(0.0))

        pltpu.async_copy(ubuf.at[pl.ds(sb, D)], out_hbm.at[row],
                         sem_out.at[so])

    def pair_step(q, _):
        r0 = 2 * q
        process_row(base + r0, r0, 0, D, 0, 0, 1)
        process_row(base + r0 + 1, r0 + 1, D, 0, 1, 1, 0)
        return 0

    lax.fori_loop(0, rows_per_w // 2, pair_step, 0)

    pltpu.make_async_copy(ubuf.at[pl.ds(0, D)],
                          out_hbm.at[base + rows_per_w - 2],
                          sem_out.at[0]).wait()
    pltpu.make_async_copy(ubuf.at[pl.ds(D, D)],
                          out_hbm.at[base + rows_per_w - 1],
                          sem_out.at[1]).wait()


def kernel(f_x, dead_latents):
    B, D = f_x.shape
    mesh = plsc.VectorSubcoreMesh(core_axis_name="c", subcore_axis_name="s",
                                  num_cores=_NC, num_subcores=_NS)
    run = pl.kernel(
        _sc_body,
        out_type=jax.ShapeDtypeStruct((B, D), jnp.float32),
        mesh=mesh,
        compiler_params=pltpu.CompilerParams(needs_layout_passes=False),
        scratch_types=[
            pltpu.VMEM((2 * D,), jnp.float32),
            pltpu.VMEM((2 * D,), jnp.float32),
            pltpu.VMEM((2 * D,), jnp.float32),
            pltpu.VMEM((D + _L,), jnp.uint32),
            pltpu.VMEM((_HIST,), jnp.int32),
            pltpu.SemaphoreType.DMA((2,)),
            pltpu.SemaphoreType.DMA((2,)),
        ],
    )
    return run(f_x, dead_latents)


# speedup vs baseline: 1.1382x; 1.0572x over previous
import numpy as np
import jax
import jax.numpy as jnp
from jax import lax
from jax.experimental import pallas as pl
from jax.experimental.pallas import tpu as pltpu
from jax.experimental.pallas import tpu_sc as plsc

_TOP_K = 512
_NC, _NS, _L = 2, 16, 16
_NW = _NC * _NS
_NB = 256
_HIST = _L * _NB


def _sc_body(f_hbm, d_hbm, out_hbm, fbuf, dbuf, cbuf, ibuf, hist,
             sem_in, sem_out):
    B, D = f_hbm.shape
    rows_per_w = B // _NW
    wid = lax.axis_index("s") * _NC + lax.axis_index("c")
    base = wid * rows_per_w
    laneseq = lax.iota(jnp.int32, _L)
    laneoff = laneseq * _NB
    ones = jnp.ones((_L,), jnp.int32)
    zeros_v = jnp.zeros((_L,), jnp.int32)
    v15 = jnp.full((_L,), _L - 1, jnp.int32)

    @plsc.parallel_loop(0, _HIST, step=_L)
    def _(i):
        hist[pl.ds(i, _L)] = zeros_v

    def scan_pass(C_v):
        init = (zeros_v, jnp.full((_L,), -1, jnp.int32), zeros_v, zeros_v)

        @plsc.parallel_loop(0, _NB, step=_L, carry=init)
        def scan(j, carry):
            run, bsel, nin, nbelow = carry
            acc = zeros_v
            for l in range(_L):
                s = pl.ds(l * _NB + j, _L)
                acc = acc + hist[s]
                hist[s] = zeros_v
            cum = jnp.cumsum(acc)
            inc = run + cum
            m = inc > C_v
            cnt = plsc.all_reduce_population_count(m)
            lane = _L - cnt
            lane_c = jnp.minimum(lane, v15)
            cnt_at = jnp.take_along_axis(acc, lane_c, axis=0)
            cum_at = jnp.take_along_axis(cum, lane_c, axis=0)
            first = jnp.logical_and(cnt > 0, bsel < 0)
            bsel = jnp.where(first, lane + j, bsel)
            nin = jnp.where(first, cnt_at, nin)
            nbelow = jnp.where(first, run + cum_at - cnt_at, nbelow)
            run = run + jnp.take_along_axis(cum, v15, axis=0)
            return run, bsel, nin, nbelow

        _, bsel, nin, nbelow = scan
        return bsel, nin, nbelow

    pltpu.async_copy(f_hbm.at[base], fbuf.at[pl.ds(0, D)], sem_in.at[0])
    pltpu.async_copy(d_hbm.at[base], dbuf.at[pl.ds(0, D)], sem_in.at[0])

    def process_row(row, r, sb, nb, si, so, ni):
        @pl.when(r + 1 < rows_per_w)
        def _():
            pltpu.async_copy(f_hbm.at[row + 1], fbuf.at[pl.ds(nb, D)],
                             sem_in.at[ni])

        pltpu.make_async_copy(f_hbm.at[row], fbuf.at[pl.ds(sb, D)],
                              sem_in.at[si]).wait()
        pltpu.make_async_copy(d_hbm.at[row], dbuf.at[pl.ds(sb, D)],
                              sem_in.at[si]).wait()

        @plsc.parallel_loop(0, D, step=_L, unroll=4)
        def _(i):
            s = pl.ds(sb + i, _L)
            f = fbuf[s]
            dd = dbuf[s]
            p = f * dd
            m = p * dd
            bits = lax.bitcast_convert_type(m, jnp.int32)
            ui = bits ^ ((bits >> 31) | jnp.int32(-2147483648))
            fbuf[s] = p
            dbuf[s] = lax.bitcast_convert_type(ui, jnp.float32)
            dig = (ui >> 24) & jnp.int32(0xFF)
            plsc.addupdate_scatter(hist, [laneoff + dig], ones)

        @pl.when(r + 1 < rows_per_w)
        def _():
            @pl.when(r >= 1)
            def _():
                pltpu.make_async_copy(dbuf.at[pl.ds(nb, D)],
                                      out_hbm.at[row - 1],
                                      sem_out.at[ni]).wait()
            pltpu.async_copy(d_hbm.at[row + 1], dbuf.at[pl.ds(nb, D)],
                             sem_in.at[ni])

        n_cur = jnp.full((_L,), D, jnp.int32)
        k_cur = jnp.full((_L,), _TOP_K, jnp.int32)

        bsel, nin, nbelow = scan_pass(n_cur - k_cur)
        k_cur = k_cur - (n_cur - nbelow - nin)
        n_cur = nin
        prefix = lax.convert_element_type(bsel, jnp.uint32)

        zf = jnp.zeros((_L,), jnp.float32)

        @plsc.parallel_loop(0, D, step=_L, unroll=4, carry=jnp.int32(0))
        def scat2(i, off, prefix=prefix):
            s = pl.ds(sb + i, _L)
            u = lax.bitcast_convert_type(dbuf[s], jnp.uint32)
            hi = u >> np.uint32(24)
            meq = hi == prefix
            mge = hi >= prefix
            dig = lax.convert_element_type(
                (u >> np.uint32(16)) & np.uint32(0xFF), jnp.int32)
            plsc.addupdate_scatter(hist, [laneoff + dig], ones, mask=meq)
            plsc.store_compressed(cbuf.at[pl.ds(off, _L)], u, mask=mge)
            plsc.store_compressed(ibuf.at[pl.ds(off, _L)], i + laneseq,
                                  mask=mge)
            dbuf[s] = zf
            return off + jnp.sum(mge.astype(jnp.int32))

        n1_s = scat2

        bsel, nin, nbelow = scan_pass(n_cur - k_cur)
        k_cur = k_cur - (n_cur - nbelow - nin)
        n_cur = nin
        prefix = (prefix << np.uint32(8)) | lax.convert_element_type(
            bsel, jnp.uint32)

        @plsc.parallel_loop(0, ((n1_s + _L - 1) // _L) * _L, step=_L)
        def _(j, prefix=prefix, n1_s=n1_s):
            u = cbuf[pl.ds(j, _L)]
            valid = (laneseq + j) < n1_s
            msk = jnp.logical_and(valid, (u >> np.uint32(16)) == prefix)
            dig = lax.convert_element_type(
                (u >> np.uint32(8)) & np.uint32(0xFF), jnp.int32)
            plsc.addupdate_scatter(hist, [laneoff + dig], ones, mask=msk)

        bsel, nin, nbelow = scan_pass(n_cur - k_cur)
        k_cur = k_cur - (n_cur - nbelow - nin)
        n_cur = nin
        prefix = (prefix << np.uint32(8)) | lax.convert_element_type(
            bsel, jnp.uint32)

        @plsc.parallel_loop(0, ((n1_s + _L - 1) // _L) * _L, step=_L)
        def _(j, prefix=prefix, n1_s=n1_s):
            u = cbuf[pl.ds(j, _L)]
            valid = (laneseq + j) < n1_s
            msk = jnp.logical_and(valid, (u >> np.uint32(8)) == prefix)
            dig = lax.convert_element_type(u & np.uint32(0xFF), jnp.int32)
            plsc.addupdate_scatter(hist, [laneoff + dig], ones, mask=msk)

        bsel, _, _ = scan_pass(n_cur - k_cur)
        thresh = (prefix << np.uint32(8)) | lax.convert_element_type(
            bsel, jnp.uint32)

        @plsc.parallel_loop(0, ((n1_s + _L - 1) // _L) * _L, step=_L)
        def _(j, thresh=thresh, n1_s=n1_s):
            u = cbuf[pl.ds(j, _L)]
            idx = ibuf[pl.ds(j, _L)]
            valid = (laneseq + j) < n1_s
            msk = jnp.logical_and(valid, u >= thresh)
            a = idx + sb
            p = plsc.load_gather(fbuf, [a], mask=msk)
            plsc.store_scatter(dbuf, [a], p, mask=msk)

        pltpu.async_copy(dbuf.at[pl.ds(sb, D)], out_hbm.at[row],
                         sem_out.at[so])

    def pair_step(q, _):
        r0 = 2 * q
        process_row(base + r0, r0, 0, D, 0, 0, 1)
        process_row(base + r0 + 1, r0 + 1, D, 0, 1, 1, 0)
        return 0

    lax.fori_loop(0, rows_per_w // 2, pair_step, 0)

    pltpu.make_async_copy(dbuf.at[pl.ds(0, D)],
                          out_hbm.at[base + rows_per_w - 2],
                          sem_out.at[0]).wait()
    pltpu.make_async_copy(dbuf.at[pl.ds(D, D)],
                          out_hbm.at[base + rows_per_w - 1],
                          sem_out.at[1]).wait()


def kernel(f_x, dead_latents):
    B, D = f_x.shape
    mesh = plsc.VectorSubcoreMesh(core_axis_name="c", subcore_axis_name="s",
                                  num_cores=_NC, num_subcores=_NS)
    run = pl.kernel(
        _sc_body,
        out_type=jax.ShapeDtypeStruct((B, D), jnp.float32),
        mesh=mesh,
        compiler_params=pltpu.CompilerParams(needs_layout_passes=False),
        scratch_types=[
            pltpu.VMEM((2 * D,), jnp.float32),
            pltpu.VMEM((2 * D,), jnp.float32),
            pltpu.VMEM((D,), jnp.uint32),
            pltpu.VMEM((D,), jnp.int32),
            pltpu.VMEM((_HIST,), jnp.int32),
            pltpu.SemaphoreType.DMA((2,)),
            pltpu.SemaphoreType.DMA((2,)),
        ],
    )
    return run(f_x, dead_latents)
